# baseline (speedup 1.0000x reference)
import dataclasses
import functools

import jax
import jax.numpy as jnp
from jax import lax
from jax.experimental import pallas as pl
from jax.experimental.pallas import tpu as pltpu
from jax.experimental.pallas import tpu_sc as plsc

NC, NS = 2, 16
NW = NC * NS
CHUNK = 256
LANES = 16


def kernel(x, weight):
    batch, seq = x.shape
    num_idx = batch * seq
    dim = weight.shape[1]
    group = 128 // dim
    b_per_w = num_idx // NW
    idx_flat = x.reshape(num_idx)
    table = weight.reshape(weight.shape[0] // group, 128)

    mesh = plsc.VectorSubcoreMesh(core_axis_name="c", subcore_axis_name="s")

    cp = pltpu.CompilerParams()
    if "needs_layout_passes" in pltpu.CompilerParams.__dataclass_fields__:
        cp = dataclasses.replace(cp, needs_layout_passes=False)

    @functools.partial(
        pl.kernel,
        mesh=mesh,
        compiler_params=cp,
        out_type=jax.ShapeDtypeStruct((num_idx, dim), jnp.float32),
        scratch_types=[
            pltpu.VMEM((CHUNK,), jnp.int32),
            pltpu.VMEM((CHUNK,), jnp.int32),
            pltpu.VMEM((CHUNK, 128), jnp.float32),
            pltpu.VMEM((CHUNK, dim), jnp.float32),
            pltpu.SemaphoreType.DMA,
        ],
    )
    def gather_kernel(table_hbm, idx_hbm, out_hbm,
                      idx_v, gidx_v, rows_v, out_v, sem):
        wid = lax.axis_index("s") * NC + lax.axis_index("c")
        base = wid * b_per_w

        @pl.loop(0, b_per_w, step=CHUNK)
        def _(off):
            pltpu.sync_copy(idx_hbm.at[pl.ds(base + off, CHUNK)], idx_v)

            @pl.loop(0, CHUNK, step=LANES)
            def _(i):
                gidx_v[pl.ds(i, LANES)] = lax.shift_right_logical(
                    idx_v[pl.ds(i, LANES)], 2)

            pltpu.async_copy(table_hbm.at[gidx_v], rows_v, sem).wait()

            @pl.loop(0, CHUNK, step=LANES)
            def _(j0):
                jvec = lax.iota(jnp.int32, LANES) + j0
                col0 = (idx_v[pl.ds(j0, LANES)] & 3) * dim
                for l in range(dim):
                    val = plsc.load_gather(rows_v, [jvec, col0 + l])
                    plsc.store_scatter(
                        out_v, [jvec, lax.full((LANES,), l, jnp.int32)], val)

            pltpu.sync_copy(out_v, out_hbm.at[pl.ds(base + off, CHUNK)])

    out = gather_kernel(table, idx_flat)
    return out.reshape(batch, seq, dim)


# speedup vs baseline: 1.6333x; 1.6333x over previous
import dataclasses
import functools

import jax
import jax.numpy as jnp
from jax import lax
from jax.experimental import pallas as pl
from jax.experimental.pallas import tpu as pltpu
from jax.experimental.pallas import tpu_sc as plsc

NC, NS = 2, 16
NW = NC * NS
CHUNK = 1024


def kernel(x, weight):
    batch, seq = x.shape
    num_idx = batch * seq
    dim = weight.shape[1]
    b_per_w = num_idx // NW
    idx_flat = x.reshape(num_idx)

    mesh = plsc.VectorSubcoreMesh(core_axis_name="c", subcore_axis_name="s")

    cp = pltpu.CompilerParams(use_tc_tiling_on_sc=False)
    if "needs_layout_passes" in pltpu.CompilerParams.__dataclass_fields__:
        cp = dataclasses.replace(cp, needs_layout_passes=False)

    @functools.partial(
        pl.kernel,
        mesh=mesh,
        compiler_params=cp,
        out_type=jax.ShapeDtypeStruct((num_idx, dim), jnp.float32),
        scratch_types=[
            pltpu.VMEM((CHUNK,), jnp.int32),
            pltpu.VMEM((CHUNK, dim), jnp.float32),
            pltpu.SemaphoreType.DMA,
        ],
    )
    def gather_kernel(table_hbm, idx_hbm, out_hbm, idx_v, rows_v, sem):
        wid = lax.axis_index("s") * NC + lax.axis_index("c")
        base = wid * b_per_w

        @pl.loop(0, b_per_w, step=CHUNK)
        def _(off):
            pltpu.sync_copy(idx_hbm.at[pl.ds(base + off, CHUNK)], idx_v)
            pltpu.async_copy(table_hbm.at[idx_v], rows_v, sem).wait()
            pltpu.sync_copy(rows_v, out_hbm.at[pl.ds(base + off, CHUNK)])

    out = gather_kernel(weight, idx_flat)
    return out.reshape(batch, seq, dim)


# speedup vs baseline: 1.6533x; 1.0122x over previous
import dataclasses
import functools

import jax
import jax.numpy as jnp
from jax import lax
from jax.experimental import pallas as pl
from jax.experimental.pallas import tpu as pltpu
from jax.experimental.pallas import tpu_sc as plsc

NC, NS = 2, 16
NW = NC * NS
CHUNK = 1280


def kernel(x, weight):
    batch, seq = x.shape
    num_idx = batch * seq
    dim = weight.shape[1]
    b_per_w = num_idx // NW
    n_ch = b_per_w // CHUNK
    idx_flat = x.reshape(num_idx)

    mesh = plsc.VectorSubcoreMesh(core_axis_name="c", subcore_axis_name="s")

    cp = pltpu.CompilerParams(use_tc_tiling_on_sc=False)
    if "needs_layout_passes" in pltpu.CompilerParams.__dataclass_fields__:
        cp = dataclasses.replace(cp, needs_layout_passes=False)

    @functools.partial(
        pl.kernel,
        mesh=mesh,
        compiler_params=cp,
        out_type=jax.ShapeDtypeStruct((num_idx, dim), jnp.float32),
        scratch_types=[
            pltpu.VMEM((CHUNK,), jnp.int32),
            pltpu.VMEM((CHUNK,), jnp.int32),
            pltpu.VMEM((CHUNK, dim), jnp.float32),
            pltpu.VMEM((CHUNK, dim), jnp.float32),
            pltpu.SemaphoreType.DMA,
            pltpu.SemaphoreType.DMA,
        ],
    )
    def gather_kernel(table_hbm, idx_hbm, out_hbm, i0, i1, r0, r1, sg0, sg1):
        wid = lax.axis_index("s") * NC + lax.axis_index("c")
        base = wid * b_per_w

        pltpu.sync_copy(idx_hbm.at[pl.ds(base, CHUNK)], i0)
        pltpu.async_copy(table_hbm.at[i0], r0, sg0)

        @pl.loop(0, n_ch, step=2)
        def _(t):
            off1 = base + (t + 1) * CHUNK
            pltpu.sync_copy(idx_hbm.at[pl.ds(off1, CHUNK)], i1)
            pltpu.async_copy(table_hbm.at[i1], r1, sg1)

            pltpu.make_async_copy(table_hbm.at[i0], r0, sg0).wait()
            pltpu.sync_copy(r0, out_hbm.at[pl.ds(base + t * CHUNK, CHUNK)])

            @pl.when(t + 2 < n_ch)
            def _():
                off2 = base + (t + 2) * CHUNK
                pltpu.sync_copy(idx_hbm.at[pl.ds(off2, CHUNK)], i0)
                pltpu.async_copy(table_hbm.at[i0], r0, sg0)

            pltpu.make_async_copy(table_hbm.at[i1], r1, sg1).wait()
            pltpu.sync_copy(r1, out_hbm.at[pl.ds(off1, CHUNK)])

    out = gather_kernel(weight, idx_flat)
    return out.reshape(batch, seq, dim)
